# Initial kernel scaffold; baseline (speedup 1.0000x reference)
import functools

import jax
import jax.numpy as jnp
from jax import lax
from jax.experimental import pallas as pl
from jax.experimental.pallas import tpu as pltpu
from jax.experimental.pallas import tpu_sc as plsc

N_NODES = 10000
N_EDGES = 160000
NFEAT = 256
NHID = 256
NCLASS = 64

NC = 2
NS = 16
CHUNK = 128
E_PAD = 163840
ROWS_PER_SUB = N_NODES // NS
_ROW_SLICES = ((0, 128), (128, 128), (256, 128), (384, 128), (512, 113))


def _make_spmm(D, CH, edge_split):
  mesh = plsc.VectorSubcoreMesh(core_axis_name="c", subcore_axis_name="s")

  @functools.partial(
      pl.kernel,
      out_type=(
          jax.ShapeDtypeStruct((N_NODES, D), jnp.float32),
          jax.ShapeDtypeStruct((N_NODES, D), jnp.float32),
      ),
      mesh=mesh,
      scratch_types=[
          pltpu.VMEM((CH, CHUNK), jnp.int32),
          pltpu.VMEM((CH, CHUNK), jnp.int32),
          pltpu.VMEM((CH, CHUNK), jnp.float32),
          pltpu.VMEM((CHUNK, D), jnp.float32),
          pltpu.VMEM((CHUNK, D), jnp.float32),
          pltpu.VMEM_SHARED((N_NODES, D), jnp.float32),
      ],
  )
  def spmm(t0, t1, src_hbm, dst_hbm, w_hbm, out0, out1,
           src_v, dst_v, w_v, rows_v, zero_v, acc):
    c = lax.axis_index("c")
    s = lax.axis_index("s")
    if edge_split:
      base = (c * NS + s) * CH
    else:
      base = s * CH

    pltpu.sync_copy(src_hbm.at[pl.ds(base, CH)], src_v)
    pltpu.sync_copy(dst_hbm.at[pl.ds(base, CH)], dst_v)
    pltpu.sync_copy(w_hbm.at[pl.ds(base, CH)], w_v)

    def zrow(i, carry):
      for f in range(D // 16):
        zero_v[i, pl.ds(f * 16, 16)] = jnp.zeros((16,), jnp.float32)
      return carry
    lax.fori_loop(0, CHUNK, zrow, 0)
    rbase = s * ROWS_PER_SUB
    for off, n in _ROW_SLICES:
      pltpu.sync_copy(zero_v.at[pl.ds(0, n)], acc.at[pl.ds(rbase + off, n)])
    plsc.subcore_barrier()

    def chunk(j, carry):
      @pl.when(c == 0)
      def _():
        pltpu.sync_copy(t0.at[src_v.at[j]], rows_v)

      @pl.when(c == 1)
      def _():
        pltpu.sync_copy(t1.at[src_v.at[j]], rows_v)

      jv = jnp.full((16,), j, jnp.int32)

      def edge(e, ecarry):
        wb = plsc.load_gather(w_v, [jv, jnp.full((16,), e, jnp.int32)])
        for f in range(D // 16):
          sl = pl.ds(f * 16, 16)
          rows_v[e, sl] = rows_v[e, sl] * wb
        return ecarry
      lax.fori_loop(0, CHUNK, edge, 0)

      pltpu.sync_copy(rows_v, acc.at[dst_v.at[j]], add=True)
      return carry
    lax.fori_loop(0, CH, chunk, 0)

    plsc.subcore_barrier()
    for off, n in _ROW_SLICES:
      sl = pl.ds(rbase + off, n)

      @pl.when(c == 0)
      def _():
        pltpu.sync_copy(acc.at[sl], out0.at[sl])

      @pl.when(c == 1)
      def _():
        pltpu.sync_copy(acc.at[sl], out1.at[sl])

  return spmm


_spmm1 = _make_spmm(NHID // 2, E_PAD // NS // CHUNK, edge_split=False)
_spmm2 = _make_spmm(NCLASS, E_PAD // (NC * NS) // CHUNK, edge_split=True)


_MM_BLOCK = 400


def _mm1_body(x_ref, w1l_ref, w1r_ref, ol_ref, or_ref):
  xb = x_ref[...]
  ol_ref[...] = jnp.dot(xb, w1l_ref[...], preferred_element_type=jnp.float32)
  or_ref[...] = jnp.dot(xb, w1r_ref[...], preferred_element_type=jnp.float32)


def _mm2_body(sl_ref, sr_ref, b1l_ref, b1r_ref, w2t_ref, w2b_ref, o_ref):
  hl = jnp.maximum(sl_ref[...] + b1l_ref[...], 0.0)
  hr = jnp.maximum(sr_ref[...] + b1r_ref[...], 0.0)
  o_ref[...] = (
      jnp.dot(hl, w2t_ref[...], preferred_element_type=jnp.float32)
      + jnp.dot(hr, w2b_ref[...], preferred_element_type=jnp.float32))


def _out_body(p0_ref, p1_ref, b2_ref, o_ref):
  z = p0_ref[...] + p1_ref[...] + b2_ref[...]
  m = jnp.max(z, axis=1, keepdims=True)
  e = jnp.exp(z - m)
  lse = jnp.log(jnp.sum(e, axis=1, keepdims=True))
  o_ref[...] = z - m - lse


def _full(shape):
  return pl.BlockSpec(shape, lambda i: (0, 0))


def _rows(shape):
  return pl.BlockSpec(shape, lambda i: (i, 0))


def kernel(x, edge_index, edge_weight, W1, b1, W2, b2):
  grid = N_NODES // _MM_BLOCK
  h1l, h1r = pl.pallas_call(
      _mm1_body,
      grid=(grid,),
      in_specs=[_rows((_MM_BLOCK, NFEAT)),
                _full((NFEAT, NHID // 2)), _full((NFEAT, NHID // 2))],
      out_specs=(_rows((_MM_BLOCK, NHID // 2)), _rows((_MM_BLOCK, NHID // 2))),
      out_shape=(jax.ShapeDtypeStruct((N_NODES, NHID // 2), jnp.float32),
                 jax.ShapeDtypeStruct((N_NODES, NHID // 2), jnp.float32)),
  )(x, W1[:, :NHID // 2], W1[:, NHID // 2:])

  src = edge_index[0].astype(jnp.int32)
  dst = edge_index[1].astype(jnp.int32)
  pad = E_PAD - N_EDGES
  src2d = jnp.pad(src, (0, pad)).reshape(E_PAD // CHUNK, CHUNK)
  dst2d = jnp.pad(dst, (0, pad)).reshape(E_PAD // CHUNK, CHUNK)
  w2d = jnp.pad(edge_weight, (0, pad)).reshape(E_PAD // CHUNK, CHUNK)

  s1l, s1r = _spmm1(h1l, h1r, src2d, dst2d, w2d)

  h2 = pl.pallas_call(
      _mm2_body,
      grid=(grid,),
      in_specs=[_rows((_MM_BLOCK, NHID // 2)), _rows((_MM_BLOCK, NHID // 2)),
                _full((1, NHID // 2)), _full((1, NHID // 2)),
                _full((NHID // 2, NCLASS)), _full((NHID // 2, NCLASS))],
      out_specs=_rows((_MM_BLOCK, NCLASS)),
      out_shape=jax.ShapeDtypeStruct((N_NODES, NCLASS), jnp.float32),
  )(s1l, s1r, b1[:NHID // 2].reshape(1, -1), b1[NHID // 2:].reshape(1, -1),
    W2[:NHID // 2], W2[NHID // 2:])

  p0, p1 = _spmm2(h2, h2, src2d, dst2d, w2d)

  out = pl.pallas_call(
      _out_body,
      grid=(grid,),
      in_specs=[_rows((_MM_BLOCK, NCLASS)), _rows((_MM_BLOCK, NCLASS)),
                _full((1, NCLASS))],
      out_specs=_rows((_MM_BLOCK, NCLASS)),
      out_shape=jax.ShapeDtypeStruct((N_NODES, NCLASS), jnp.float32),
  )(p0, p1, b2.reshape(1, -1))
  return out


# speedup vs baseline: 2.4456x; 2.4456x over previous
import functools

import jax
import jax.numpy as jnp
from jax import lax
from jax.experimental import pallas as pl
from jax.experimental.pallas import tpu as pltpu
from jax.experimental.pallas import tpu_sc as plsc

N_NODES = 10000
N_EDGES = 160000
NFEAT = 256
NHID = 256
NCLASS = 64

NC = 2
NS = 16
CHUNK = 128
E_PAD = 163840
ROWS_PER_SUB = 624
_ROW_SLICES = ((0, 128), (128, 128), (256, 128), (384, 128), (496, 128))


def _make_spmm(D, CH, edge_split):
  mesh = plsc.VectorSubcoreMesh(core_axis_name="c", subcore_axis_name="s")

  @functools.partial(
      pl.kernel,
      out_type=(
          jax.ShapeDtypeStruct((N_NODES, D), jnp.float32),
          jax.ShapeDtypeStruct((N_NODES, D), jnp.float32),
      ),
      mesh=mesh,
      scratch_types=[
          pltpu.VMEM((CH, CHUNK), jnp.int32),
          pltpu.VMEM((CH, CHUNK), jnp.int32),
          pltpu.VMEM((CH, CHUNK), jnp.float32),
          pltpu.VMEM((CHUNK, D), jnp.float32),
          pltpu.VMEM_SHARED((N_NODES, D), jnp.float32),
      ],
  )
  def spmm(t0, t1, src_hbm, dst_hbm, w_hbm, out0, out1,
           src_v, dst_v, w_v, rows_v, acc):
    c = lax.axis_index("c")
    s = lax.axis_index("s")
    if edge_split:
      base = (c * NS + s) * CH
    else:
      base = s * CH

    pltpu.sync_copy(src_hbm.at[pl.ds(base, CH)], src_v)
    pltpu.sync_copy(dst_hbm.at[pl.ds(base, CH)], dst_v)
    pltpu.sync_copy(w_hbm.at[pl.ds(base, CH)], w_v)

    def zrow(i, carry):
      for f in range(D // 16):
        rows_v[i, pl.ds(f * 16, 16)] = jnp.zeros((16,), jnp.float32)
      return carry
    lax.fori_loop(0, CHUNK, zrow, 0)
    rbase = s * ROWS_PER_SUB
    for off, n in _ROW_SLICES:
      pltpu.sync_copy(rows_v.at[pl.ds(0, n)], acc.at[pl.ds(rbase + off, n)])

    @pl.when(s == NS - 1)
    def _():
      pltpu.sync_copy(rows_v.at[pl.ds(0, 16)],
                      acc.at[pl.ds(NS * ROWS_PER_SUB, 16)])
    plsc.subcore_barrier()

    def chunk(j, carry):
      @pl.when(c == 0)
      def _():
        pltpu.sync_copy(t0.at[src_v.at[j]], rows_v)

      @pl.when(c == 1)
      def _():
        pltpu.sync_copy(t1.at[src_v.at[j]], rows_v)

      def edge16(g, ecarry):
        w16 = w_v[j, pl.ds(g * 16, 16)]
        for e0 in range(16):
          wb = w16[e0]
          e = g * 16 + e0
          for f in range(D // 16):
            sl = pl.ds(f * 16, 16)
            rows_v[e, sl] = rows_v[e, sl] * wb
        return ecarry
      lax.fori_loop(0, CHUNK // 16, edge16, 0)

      pltpu.sync_copy(rows_v, acc.at[dst_v.at[j]], add=True)
      return carry
    lax.fori_loop(0, CH, chunk, 0)

    plsc.subcore_barrier()
    tail = pl.ds(NS * ROWS_PER_SUB, 16)
    for off, n in _ROW_SLICES:
      sl = pl.ds(rbase + off, n)

      @pl.when(c == 0)
      def _():
        pltpu.sync_copy(acc.at[sl], out0.at[sl])

      @pl.when(c == 1)
      def _():
        pltpu.sync_copy(acc.at[sl], out1.at[sl])

    @pl.when(jnp.logical_and(s == NS - 1, c == 0))
    def _():
      pltpu.sync_copy(acc.at[tail], out0.at[tail])

    @pl.when(jnp.logical_and(s == NS - 1, c == 1))
    def _():
      pltpu.sync_copy(acc.at[tail], out1.at[tail])

  return spmm


_spmm1 = _make_spmm(NHID // 2, E_PAD // NS // CHUNK, edge_split=False)
_spmm2 = _make_spmm(128, E_PAD // (NC * NS) // CHUNK, edge_split=True)


_MM_BLOCK = 400


def _mm1_body(x_ref, w1l_ref, w1r_ref, ol_ref, or_ref):
  xb = x_ref[...]
  ol_ref[...] = jnp.dot(xb, w1l_ref[...], preferred_element_type=jnp.float32)
  or_ref[...] = jnp.dot(xb, w1r_ref[...], preferred_element_type=jnp.float32)


def _mm2_body(sl_ref, sr_ref, b1l_ref, b1r_ref, w2t_ref, w2b_ref, o_ref):
  hl = jnp.maximum(sl_ref[...] + b1l_ref[...], 0.0)
  hr = jnp.maximum(sr_ref[...] + b1r_ref[...], 0.0)
  o_ref[...] = (
      jnp.dot(hl, w2t_ref[...], preferred_element_type=jnp.float32)
      + jnp.dot(hr, w2b_ref[...], preferred_element_type=jnp.float32))


def _out_body(p0_ref, p1_ref, b2_ref, o_ref):
  zp = p0_ref[...] + p1_ref[...]
  z = zp[:, :NCLASS] + b2_ref[...]
  m = jnp.max(z, axis=1, keepdims=True)
  e = jnp.exp(z - m)
  lse = jnp.log(jnp.sum(e, axis=1, keepdims=True))
  o_ref[...] = z - m - lse


def _full(shape):
  return pl.BlockSpec(shape, lambda i: (0, 0))


def _rows(shape):
  return pl.BlockSpec(shape, lambda i: (i, 0))


def kernel(x, edge_index, edge_weight, W1, b1, W2, b2):
  grid = N_NODES // _MM_BLOCK
  h1l, h1r = pl.pallas_call(
      _mm1_body,
      grid=(grid,),
      in_specs=[_rows((_MM_BLOCK, NFEAT)),
                _full((NFEAT, NHID // 2)), _full((NFEAT, NHID // 2))],
      out_specs=(_rows((_MM_BLOCK, NHID // 2)), _rows((_MM_BLOCK, NHID // 2))),
      out_shape=(jax.ShapeDtypeStruct((N_NODES, NHID // 2), jnp.float32),
                 jax.ShapeDtypeStruct((N_NODES, NHID // 2), jnp.float32)),
  )(x, W1[:, :NHID // 2], W1[:, NHID // 2:])

  src = edge_index[0].astype(jnp.int32)
  dst = edge_index[1].astype(jnp.int32)
  pad = E_PAD - N_EDGES
  src2d = jnp.pad(src, (0, pad)).reshape(E_PAD // CHUNK, CHUNK)
  dst2d = jnp.pad(dst, (0, pad)).reshape(E_PAD // CHUNK, CHUNK)
  w2d = jnp.pad(edge_weight, (0, pad)).reshape(E_PAD // CHUNK, CHUNK)

  s1l, s1r = _spmm1(h1l, h1r, src2d, dst2d, w2d)

  W2p = jnp.pad(W2, ((0, 0), (0, 128 - NCLASS)))
  h2 = pl.pallas_call(
      _mm2_body,
      grid=(grid,),
      in_specs=[_rows((_MM_BLOCK, NHID // 2)), _rows((_MM_BLOCK, NHID // 2)),
                _full((1, NHID // 2)), _full((1, NHID // 2)),
                _full((NHID // 2, 128)), _full((NHID // 2, 128))],
      out_specs=_rows((_MM_BLOCK, 128)),
      out_shape=jax.ShapeDtypeStruct((N_NODES, 128), jnp.float32),
  )(s1l, s1r, b1[:NHID // 2].reshape(1, -1), b1[NHID // 2:].reshape(1, -1),
    W2p[:NHID // 2], W2p[NHID // 2:])

  p0, p1 = _spmm2(h2, h2, src2d, dst2d, w2d)

  out = pl.pallas_call(
      _out_body,
      grid=(grid,),
      in_specs=[_rows((_MM_BLOCK, 128)), _rows((_MM_BLOCK, 128)),
                _full((1, NCLASS))],
      out_specs=_rows((_MM_BLOCK, NCLASS)),
      out_shape=jax.ShapeDtypeStruct((N_NODES, NCLASS), jnp.float32),
  )(p0, p1, b2.reshape(1, -1))
  return out


# speedup vs baseline: 5.7062x; 2.3333x over previous
import functools

import jax
import jax.numpy as jnp
from jax import lax
from jax.experimental import pallas as pl
from jax.experimental.pallas import tpu as pltpu
from jax.experimental.pallas import tpu_sc as plsc

N_NODES = 10000
N_EDGES = 160000
NFEAT = 256
NHID = 256
NCLASS = 64

NC = 2
NS = 16
EPC = 64
E_PAD = 163840
ROWS_PER_SUB = 624
_ROW_SLICES = tuple((k * 64, 64) for k in range(9)) + ((576, 48),)

def _make_spmm(D, CH, edge_split, nwin):
  mesh = plsc.VectorSubcoreMesh(core_axis_name="c", subcore_axis_name="s")
  WCH = CH // nwin

  @functools.partial(
      pl.kernel,
      out_type=(
          jax.ShapeDtypeStruct((N_NODES, D), jnp.float32),
          jax.ShapeDtypeStruct((N_NODES, D), jnp.float32),
      ),
      mesh=mesh,
      scratch_types=[
          pltpu.VMEM((WCH, EPC), jnp.int32),
          pltpu.VMEM((WCH, EPC), jnp.int32),
          pltpu.VMEM((WCH, EPC), jnp.float32),
          [pltpu.VMEM((EPC, D), jnp.float32)] * 2,
          [pltpu.VMEM((EPC, D), jnp.float32)] * 2,
          [pltpu.SemaphoreType.DMA] * 2,
          [pltpu.SemaphoreType.DMA] * 2,
          pltpu.VMEM_SHARED((N_NODES, D), jnp.float32),
      ],
  )
  def spmm(t0, t1, src_hbm, dst_hbm, w_hbm, out0, out1,
           src_v, dst_v, w_v, rows, cmp, gsem, ssem, acc):
    c = lax.axis_index("c")
    s = lax.axis_index("s")
    if edge_split:
      base = (c * NS + s) * CH
    else:
      base = s * CH

    def zrow(i, carry):
      for f in range(D // 16):
        cmp[0][i, pl.ds(f * 16, 16)] = jnp.zeros((16,), jnp.float32)
      return carry
    lax.fori_loop(0, 64, zrow, 0)
    rbase = s * ROWS_PER_SUB
    for off, n in _ROW_SLICES:
      pltpu.sync_copy(cmp[0].at[pl.ds(0, n)], acc.at[pl.ds(rbase + off, n)])

    @pl.when(s == NS - 1)
    def _():
      pltpu.sync_copy(cmp[0].at[pl.ds(0, 16)],
                      acc.at[pl.ds(NS * ROWS_PER_SUB, 16)])
    plsc.subcore_barrier()

    def gather_start(jj, b):
      @pl.when(c == 0)
      def _():
        pltpu.async_copy(t0.at[src_v.at[jj]], rows[b], gsem[b])

      @pl.when(c == 1)
      def _():
        pltpu.async_copy(t1.at[src_v.at[jj]], rows[b], gsem[b])

    def gather_wait(b):
      pltpu.make_async_copy(t0.at[src_v.at[0]], rows[b], gsem[b]).wait()

    def scale(b, jj):
      def edge16(g, ecarry):
        w16 = w_v[jj, pl.ds(g * 16, 16)]
        for e0 in range(16):
          wb = w16[e0]
          e = g * 16 + e0
          for q in range(D // 16):
            sl = pl.ds(q * 16, 16)
            cmp[b][e, sl] = rows[b][e, sl] * wb
        return ecarry
      lax.fori_loop(0, EPC // 16, edge16, 0)

    def scatter_start(jj, b):
      pltpu.async_copy(cmp[b], acc.at[dst_v.at[jj]], ssem[b], add=True)

    def scatter_wait(b):
      pltpu.make_async_copy(cmp[b], acc.at[dst_v.at[0]], ssem[b]).wait()

    def window(win, wcarry):
      wbase = base + win * WCH
      pltpu.sync_copy(src_hbm.at[pl.ds(wbase, WCH)], src_v)
      pltpu.sync_copy(dst_hbm.at[pl.ds(wbase, WCH)], dst_v)
      pltpu.sync_copy(w_hbm.at[pl.ds(wbase, WCH)], w_v)
      gather_start(0, 0)
      gather_start(1, 1)

      def steady(i, carry):
        for b in range(2):
          j = 2 * i + b

          @pl.when(i > 0)
          def _():
            scatter_wait(b)
          gather_wait(b)
          scale(b, j)
          gather_start(jnp.minimum(j + 2, WCH - 1), b)
          scatter_start(j, b)
        return carry
      lax.fori_loop(0, WCH // 2, steady, 0)
      for b in range(2):
        scatter_wait(b)
        gather_wait(b)
      return wcarry
    lax.fori_loop(0, nwin, window, 0)

    plsc.subcore_barrier()
    tail = pl.ds(NS * ROWS_PER_SUB, 16)
    for off, n in _ROW_SLICES:
      sl = pl.ds(rbase + off, n)

      @pl.when(c == 0)
      def _():
        pltpu.sync_copy(acc.at[sl], out0.at[sl])

      @pl.when(c == 1)
      def _():
        pltpu.sync_copy(acc.at[sl], out1.at[sl])

    @pl.when(jnp.logical_and(s == NS - 1, c == 0))
    def _():
      pltpu.sync_copy(acc.at[tail], out0.at[tail])

    @pl.when(jnp.logical_and(s == NS - 1, c == 1))
    def _():
      pltpu.sync_copy(acc.at[tail], out1.at[tail])

  return spmm


_spmm1 = _make_spmm(NHID // 2, E_PAD // NS // EPC, edge_split=False, nwin=10)
_spmm2 = _make_spmm(128, E_PAD // (NC * NS) // EPC, edge_split=True, nwin=5)


_MM_BLOCK = 400


def _mm1_body(x_ref, w1l_ref, w1r_ref, ol_ref, or_ref):
  xb = x_ref[...].astype(jnp.bfloat16)
  ol_ref[...] = jnp.dot(xb, w1l_ref[...], preferred_element_type=jnp.float32)
  or_ref[...] = jnp.dot(xb, w1r_ref[...], preferred_element_type=jnp.float32)


def _mm2_body(sl_ref, sr_ref, b1l_ref, b1r_ref, w2t_ref, w2b_ref, o_ref):
  hl = jnp.maximum(sl_ref[...] + b1l_ref[...], 0.0).astype(jnp.bfloat16)
  hr = jnp.maximum(sr_ref[...] + b1r_ref[...], 0.0).astype(jnp.bfloat16)
  o_ref[...] = (
      jnp.dot(hl, w2t_ref[...], preferred_element_type=jnp.float32)
      + jnp.dot(hr, w2b_ref[...], preferred_element_type=jnp.float32))


def _out_body(p0_ref, p1_ref, b2_ref, o_ref):
  zp = p0_ref[...] + p1_ref[...]
  z = zp[:, :NCLASS] + b2_ref[...]
  m = jnp.max(z, axis=1, keepdims=True)
  e = jnp.exp(z - m)
  lse = jnp.log(jnp.sum(e, axis=1, keepdims=True))
  o_ref[...] = z - m - lse


def _full(shape):
  return pl.BlockSpec(shape, lambda i: (0, 0))


def _rows(shape):
  return pl.BlockSpec(shape, lambda i: (i, 0))


def kernel(x, edge_index, edge_weight, W1, b1, W2, b2):
  grid = N_NODES // _MM_BLOCK
  W1p = W1.astype(jnp.bfloat16)
  h1l, h1r = pl.pallas_call(
      _mm1_body,
      grid=(grid,),
      in_specs=[_rows((_MM_BLOCK, NFEAT)),
                _full((NFEAT, NHID // 2)), _full((NFEAT, NHID // 2))],
      out_specs=(_rows((_MM_BLOCK, NHID // 2)), _rows((_MM_BLOCK, NHID // 2))),
      out_shape=(jax.ShapeDtypeStruct((N_NODES, NHID // 2), jnp.float32),
                 jax.ShapeDtypeStruct((N_NODES, NHID // 2), jnp.float32)),
  )(x, W1p[:, :NHID // 2], W1p[:, NHID // 2:])

  src = edge_index[0].astype(jnp.int32)
  dst = edge_index[1].astype(jnp.int32)
  pad = E_PAD - N_EDGES
  pad_idx = jnp.arange(pad, dtype=jnp.int32) % N_NODES
  src2d = jnp.concatenate([src, pad_idx]).reshape(E_PAD // EPC, EPC)
  dst2d = jnp.concatenate([dst, pad_idx]).reshape(E_PAD // EPC, EPC)
  w2d = jnp.pad(edge_weight, (0, pad)).reshape(E_PAD // EPC, EPC)

  s1l, s1r = _spmm1(h1l, h1r, src2d, dst2d, w2d)

  W2p = jnp.pad(W2, ((0, 0), (0, 128 - NCLASS))).astype(jnp.bfloat16)
  h2 = pl.pallas_call(
      _mm2_body,
      grid=(grid,),
      in_specs=[_rows((_MM_BLOCK, NHID // 2)), _rows((_MM_BLOCK, NHID // 2)),
                _full((1, NHID // 2)), _full((1, NHID // 2)),
                _full((NHID // 2, 128)), _full((NHID // 2, 128))],
      out_specs=_rows((_MM_BLOCK, 128)),
      out_shape=jax.ShapeDtypeStruct((N_NODES, 128), jnp.float32),
  )(s1l, s1r, b1[:NHID // 2].reshape(1, -1), b1[NHID // 2:].reshape(1, -1),
    W2p[:NHID // 2], W2p[NHID // 2:])

  p0, p1 = _spmm2(h2, h2, src2d, dst2d, w2d)

  out = pl.pallas_call(
      _out_body,
      grid=(grid,),
      in_specs=[_rows((_MM_BLOCK, 128)), _rows((_MM_BLOCK, 128)),
                _full((1, NCLASS))],
      out_specs=_rows((_MM_BLOCK, NCLASS)),
      out_shape=jax.ShapeDtypeStruct((N_NODES, NCLASS), jnp.float32),
  )(p0, p1, b2.reshape(1, -1))
  return out


# speedup vs baseline: 6.8467x; 1.1999x over previous
import functools

import jax
import jax.numpy as jnp
from jax import lax
from jax.experimental import pallas as pl
from jax.experimental.pallas import tpu as pltpu
from jax.experimental.pallas import tpu_sc as plsc

N_NODES = 10000
N_EDGES = 160000
NFEAT = 256
NHID = 256
NCLASS = 64

NC = 2
NS = 16
EPC = 128
E_PAD = 163840
ROWS_PER_SUB = 624
_ROW_SLICES = tuple((k * 64, 64) for k in range(9)) + ((576, 48),)

def _make_spmm(DT, DO, CH, edge_split, nwin):
  mesh = plsc.VectorSubcoreMesh(core_axis_name="c", subcore_axis_name="s")
  WCH = CH // nwin
  split = DO < DT

  scratch = [
      pltpu.VMEM((WCH, EPC), jnp.int32),
      pltpu.VMEM((WCH, EPC), jnp.int32),
      pltpu.VMEM((WCH, EPC), jnp.float32),
      [pltpu.VMEM((EPC, DT), jnp.float32)] * 2,
      [pltpu.SemaphoreType.DMA] * 2,
      [pltpu.SemaphoreType.DMA] * 2,
      pltpu.VMEM_SHARED((N_NODES, DO), jnp.float32),
  ]
  if split:
    scratch.append([pltpu.VMEM((EPC, DO), jnp.float32)] * 2)

  @functools.partial(
      pl.kernel,
      out_type=(
          jax.ShapeDtypeStruct((N_NODES, DO), jnp.float32),
          jax.ShapeDtypeStruct((N_NODES, DO), jnp.float32),
      ),
      mesh=mesh,
      scratch_types=scratch,
  )
  def spmm(t0, t1, src_hbm, dst_hbm, w_hbm, out0, out1,
           src_v, dst_v, w_v, rows, gsem, ssem, acc, *rest):
    cmp = rest[0] if split else rows
    c = lax.axis_index("c")
    s = lax.axis_index("s")
    if edge_split:
      base = (c * NS + s) * CH
    else:
      base = s * CH

    def zrow(i, carry):
      for f in range(DO // 16):
        cmp[0][i, pl.ds(f * 16, 16)] = jnp.zeros((16,), jnp.float32)
      return carry
    lax.fori_loop(0, 64, zrow, 0)
    rbase = s * ROWS_PER_SUB
    for off, n in _ROW_SLICES:
      pltpu.sync_copy(cmp[0].at[pl.ds(0, n)], acc.at[pl.ds(rbase + off, n)])

    @pl.when(s == NS - 1)
    def _():
      pltpu.sync_copy(cmp[0].at[pl.ds(0, 16)],
                      acc.at[pl.ds(NS * ROWS_PER_SUB, 16)])
    plsc.subcore_barrier()

    def gather_start(jj, b):
      @pl.when(c == 0)
      def _():
        pltpu.async_copy(t0.at[src_v.at[jj]], rows[b], gsem[b])

      @pl.when(c == 1)
      def _():
        pltpu.async_copy(t1.at[src_v.at[jj]], rows[b], gsem[b])

    def gather_wait(b):
      pltpu.make_async_copy(t0.at[src_v.at[0]], rows[b], gsem[b]).wait()

    def scale(b, jj):
      def edge16(g, ecarry):
        w16 = w_v[jj, pl.ds(g * 16, 16)]
        for e0 in range(16):
          wb = w16[e0]
          e = g * 16 + e0
          for q in range(DO // 16):
            sl = pl.ds(q * 16, 16)
            cmp[b][e, sl] = rows[b][e, sl] * wb
        return ecarry
      lax.fori_loop(0, EPC // 16, edge16, 0)

    def scatter_start(jj, b):
      pltpu.async_copy(cmp[b], acc.at[dst_v.at[jj]], ssem[b], add=True)

    def scatter_wait(b):
      pltpu.make_async_copy(cmp[b], acc.at[dst_v.at[0]], ssem[b]).wait()

    for win in range(nwin):
      wbase = base + win * WCH
      pltpu.sync_copy(src_hbm.at[pl.ds(wbase, WCH)], src_v)
      pltpu.sync_copy(dst_hbm.at[pl.ds(wbase, WCH)], dst_v)
      pltpu.sync_copy(w_hbm.at[pl.ds(wbase, WCH)], w_v)
      gather_start(0, 0)
      gather_start(1, 1)

      def pair(i, carry):
        for b in range(2):
          j = 2 * i + b
          gather_wait(b)
          if split:
            @pl.when(i > 0)
            def _():
              scatter_wait(b)
            scale(b, j)
            gather_start(jnp.minimum(j + 2, WCH - 1), b)
            scatter_start(j, b)
          else:
            scale(b, j)
            scatter_start(j, b)
        if not split:
          for b in range(2):
            scatter_wait(b)
            gather_start(jnp.minimum(2 * i + b + 2, WCH - 1), b)
        return carry
      lax.fori_loop(0, WCH // 2, pair, 0)
      for b in range(2):
        if split:
          scatter_wait(b)
        gather_wait(b)

    plsc.subcore_barrier()
    tail = pl.ds(NS * ROWS_PER_SUB, 16)
    for off, n in _ROW_SLICES:
      sl = pl.ds(rbase + off, n)

      @pl.when(c == 0)
      def _():
        pltpu.sync_copy(acc.at[sl], out0.at[sl])

      @pl.when(c == 1)
      def _():
        pltpu.sync_copy(acc.at[sl], out1.at[sl])

    @pl.when(jnp.logical_and(s == NS - 1, c == 0))
    def _():
      pltpu.sync_copy(acc.at[tail], out0.at[tail])

    @pl.when(jnp.logical_and(s == NS - 1, c == 1))
    def _():
      pltpu.sync_copy(acc.at[tail], out1.at[tail])

  return spmm


_spmm1 = _make_spmm(NHID // 2, NHID // 2, E_PAD // NS // EPC,
                    edge_split=False, nwin=2)
_spmm2 = _make_spmm(128, NCLASS, E_PAD // (NC * NS) // EPC,
                    edge_split=True, nwin=1)


_MM_BLOCK = 400


def _mm1_body(x_ref, w1l_ref, w1r_ref, ol_ref, or_ref):
  xb = x_ref[...].astype(jnp.bfloat16)
  ol_ref[...] = jnp.dot(xb, w1l_ref[...], preferred_element_type=jnp.float32)
  or_ref[...] = jnp.dot(xb, w1r_ref[...], preferred_element_type=jnp.float32)


def _mm2_body(sl_ref, sr_ref, b1l_ref, b1r_ref, w2t_ref, w2b_ref, o_ref):
  hl = jnp.maximum(sl_ref[...] + b1l_ref[...], 0.0).astype(jnp.bfloat16)
  hr = jnp.maximum(sr_ref[...] + b1r_ref[...], 0.0).astype(jnp.bfloat16)
  o_ref[...] = (
      jnp.dot(hl, w2t_ref[...], preferred_element_type=jnp.float32)
      + jnp.dot(hr, w2b_ref[...], preferred_element_type=jnp.float32))


def _out_body(p0_ref, p1_ref, b2_ref, o_ref):
  z = p0_ref[...] + p1_ref[...] + b2_ref[...]
  m = jnp.max(z, axis=1, keepdims=True)
  e = jnp.exp(z - m)
  lse = jnp.log(jnp.sum(e, axis=1, keepdims=True))
  o_ref[...] = z - m - lse


def _full(shape):
  return pl.BlockSpec(shape, lambda i: (0, 0))


def _rows(shape):
  return pl.BlockSpec(shape, lambda i: (i, 0))


def kernel(x, edge_index, edge_weight, W1, b1, W2, b2):
  grid = N_NODES // _MM_BLOCK
  W1p = W1.astype(jnp.bfloat16)
  h1l, h1r = pl.pallas_call(
      _mm1_body,
      grid=(grid,),
      in_specs=[_rows((_MM_BLOCK, NFEAT)),
                _full((NFEAT, NHID // 2)), _full((NFEAT, NHID // 2))],
      out_specs=(_rows((_MM_BLOCK, NHID // 2)), _rows((_MM_BLOCK, NHID // 2))),
      out_shape=(jax.ShapeDtypeStruct((N_NODES, NHID // 2), jnp.float32),
                 jax.ShapeDtypeStruct((N_NODES, NHID // 2), jnp.float32)),
  )(x, W1p[:, :NHID // 2], W1p[:, NHID // 2:])

  src = edge_index[0].astype(jnp.int32)
  dst = edge_index[1].astype(jnp.int32)
  pad = E_PAD - N_EDGES
  pad_idx = jnp.arange(pad, dtype=jnp.int32) % N_NODES
  src2d = jnp.concatenate([src, pad_idx]).reshape(E_PAD // EPC, EPC)
  dst2d = jnp.concatenate([dst, pad_idx]).reshape(E_PAD // EPC, EPC)
  w2d = jnp.pad(edge_weight, (0, pad)).reshape(E_PAD // EPC, EPC)

  s1l, s1r = _spmm1(h1l, h1r, src2d, dst2d, w2d)

  W2p = jnp.pad(W2, ((0, 0), (0, 128 - NCLASS))).astype(jnp.bfloat16)
  h2 = pl.pallas_call(
      _mm2_body,
      grid=(grid,),
      in_specs=[_rows((_MM_BLOCK, NHID // 2)), _rows((_MM_BLOCK, NHID // 2)),
                _full((1, NHID // 2)), _full((1, NHID // 2)),
                _full((NHID // 2, 128)), _full((NHID // 2, 128))],
      out_specs=_rows((_MM_BLOCK, 128)),
      out_shape=jax.ShapeDtypeStruct((N_NODES, 128), jnp.float32),
  )(s1l, s1r, b1[:NHID // 2].reshape(1, -1), b1[NHID // 2:].reshape(1, -1),
    W2p[:NHID // 2], W2p[NHID // 2:])

  p0, p1 = _spmm2(h2, h2, src2d, dst2d, w2d)

  out = pl.pallas_call(
      _out_body,
      grid=(grid,),
      in_specs=[_rows((_MM_BLOCK, NCLASS)), _rows((_MM_BLOCK, NCLASS)),
                _full((1, NCLASS))],
      out_specs=_rows((_MM_BLOCK, NCLASS)),
      out_shape=jax.ShapeDtypeStruct((N_NODES, NCLASS), jnp.float32),
  )(p0, p1, b2.reshape(1, -1))
  return out
